# baseline (speedup 1.0000x reference)
import functools

import jax
import jax.numpy as jnp
from jax.experimental import pallas as pl


def _mm_body(x_ref, w_ref, o_ref):
    o_ref[...] = jnp.dot(x_ref[...], w_ref[...], preferred_element_type=jnp.float32)


def _matmul(x, w):
    m, k = x.shape
    k2, n = w.shape
    bm = 512
    grid = (pl.cdiv(m, bm),)
    return pl.pallas_call(
        _mm_body,
        grid=grid,
        in_specs=[pl.BlockSpec((bm, k), lambda i: (i, 0)),
                  pl.BlockSpec((k, n), lambda i: (0, 0))],
        out_specs=pl.BlockSpec((bm, n), lambda i: (i, 0)),
        out_shape=jax.ShapeDtypeStruct((m, n), jnp.float32),
    )(x, w)


def _gat_layer(x, edge_index, W, attn_l, attn_r, bias, num_heads, out_feats):
    n = x.shape[0]
    src = edge_index[0]
    dst = edge_index[1]
    feat = _matmul(x, W).reshape(n, num_heads, out_feats)
    el = jnp.sum(feat * attn_l[None, :, :], axis=-1)
    er = jnp.sum(feat * attn_r[None, :, :], axis=-1)
    e = jax.nn.leaky_relu(el[src] + er[dst], negative_slope=0.2)
    e_max = jax.ops.segment_max(e, dst, num_segments=n)
    e_max = jnp.where(jnp.isfinite(e_max), e_max, 0.0)
    alpha = jnp.exp(e - e_max[dst])
    denom = jax.ops.segment_sum(alpha, dst, num_segments=n)
    alpha = alpha / denom[dst]
    msg = feat[src] * alpha[:, :, None]
    rst = jax.ops.segment_sum(msg, dst, num_segments=n)
    return rst + bias.reshape(1, num_heads, out_feats)


def kernel(x, W1, attn_l1, attn_r1, bias1, W2, attn_l2, attn_r2, bias2, edge_index, graph_len):
    n = x.shape[0]
    h = _gat_layer(x, edge_index, W1, attn_l1, attn_r1, bias1, 4, 256)
    h = jax.nn.relu(h).reshape(n, 4 * 256)
    h = _gat_layer(h, edge_index, W2, attn_l2, attn_r2, bias2, 1, 256)
    graph_output = jax.nn.relu(h).squeeze(1)
    graph_embedding = jnp.sum(graph_output, axis=1)
    return (graph_embedding, graph_output)


# speedup vs baseline: 9.5308x; 9.5308x over previous
import functools

import jax
import jax.numpy as jnp
from jax import lax
from jax.experimental import pallas as pl
from jax.experimental.pallas import tpu as pltpu
from jax.experimental.pallas import tpu_sc as plsc

N = 10000
E = 160000
HID = 256

EW = E // 16
EWP = 10112
NGRP = (EW + 16) // 16
PAD0 = EW
MLCAP = 10112
NP = 10240
NPAD = 10240
CH = 32
BIG = 1 << 20

NB = 3328
NPASS = 2
OUT_ROWS = 2 * NPASS * NB


def _proj_body(nheads, x_ref, w_ref, al_ref, ar_ref, ft_ref, el_ref, er_ref):
    x = x_ref[...]
    w = w_ref[...]
    feat = jnp.dot(x, w, preferred_element_type=jnp.float32)
    hd = w.shape[1]
    for h in range(nheads):
        ft_ref[h, :, :] = feat[:, h * HID:(h + 1) * HID]
    row = lax.broadcasted_iota(jnp.int32, (nheads, hd), 0)
    col = lax.broadcasted_iota(jnp.int32, (nheads, hd), 1)
    blk = col // HID
    dn = (([1], [1]), ([], []))
    alf = jnp.where(row == blk, jnp.broadcast_to(al_ref[...], (nheads, hd)), 0.0)
    arf = jnp.where(row == blk, jnp.broadcast_to(ar_ref[...], (nheads, hd)), 0.0)
    i = pl.program_id(0)
    bm = x.shape[0]
    el_ref[:, pl.ds(i * bm, bm)] = lax.dot_general(
        alf, feat, dn, preferred_element_type=jnp.float32)
    er_ref[:, pl.ds(i * bm, bm)] = lax.dot_general(
        arf, feat, dn, preferred_element_type=jnp.float32)


def _proj(x, w, attn_l, attn_r, nheads):
    n = x.shape[0]
    hd = w.shape[1]
    bm = 1024
    grid = (n // bm,)
    return pl.pallas_call(
        functools.partial(_proj_body, nheads),
        grid=grid,
        in_specs=[
            pl.BlockSpec((bm, x.shape[1]), lambda i: (i, 0)),
            pl.BlockSpec((x.shape[1], hd), lambda i: (0, 0)),
            pl.BlockSpec((1, hd), lambda i: (0, 0)),
            pl.BlockSpec((1, hd), lambda i: (0, 0)),
        ],
        out_specs=[
            pl.BlockSpec((nheads, bm, HID), lambda i: (0, i, 0)),
            pl.BlockSpec((nheads, n), lambda i: (0, 0)),
            pl.BlockSpec((nheads, n), lambda i: (0, 0)),
        ],
        out_shape=[
            jax.ShapeDtypeStruct((nheads, n, HID), jnp.float32),
            jax.ShapeDtypeStruct((nheads, n), jnp.float32),
            jax.ShapeDtypeStruct((nheads, n), jnp.float32),
        ],
    )(x, w, attn_l, attn_r)


def _mx_body(el_ref, g_ref):
    gmax = jnp.max(el_ref[...], axis=1, keepdims=True)
    g_ref[...] = jnp.broadcast_to(gmax, g_ref.shape)


def _gmax(elT):
    h, n = elT.shape
    return pl.pallas_call(
        _mx_body,
        out_shape=jax.ShapeDtypeStruct((h, 16), jnp.float32),
    )(elT)


def _udiv_body(nheads, u_ref, d_ref, b_ref, h_ref):
    u = u_ref[...]
    d = d_ref[...]
    b = b_ref[...]
    segs = []
    for h in range(nheads):
        dh = d[:, h:h + 1]
        segs.append(jnp.where(dh > 0.0, u[h] / dh, 0.0))
    hh = jnp.concatenate(segs, axis=1) if nheads > 1 else segs[0]
    h_ref[...] = jnp.maximum(hh + b, 0.0)


def _udiv_relu(u_pad, den_pad, bias2d, nheads):
    bm = 1024
    grid = (NP // bm,)
    return pl.pallas_call(
        functools.partial(_udiv_body, nheads),
        grid=grid,
        in_specs=[
            pl.BlockSpec((nheads, bm, HID), lambda i: (0, i, 0)),
            pl.BlockSpec((bm, 16), lambda i: (i, 0)),
            pl.BlockSpec((1, nheads * HID), lambda i: (0, 0)),
        ],
        out_specs=pl.BlockSpec((bm, nheads * HID), lambda i: (i, 0)),
        out_shape=jax.ShapeDtypeStruct((NP, nheads * HID), jnp.float32),
    )(u_pad, den_pad, bias2d)


def _fin_body(u_ref, d_ref, b_ref, g_ref, ge_ref):
    u = u_ref[...][0]
    dh = d_ref[...][:, 0:1]
    g = jnp.where(dh > 0.0, u / dh, 0.0)
    g = jnp.maximum(g + b_ref[...], 0.0)
    g_ref[...] = g
    ge_ref[...] = jnp.sum(g, axis=1, keepdims=True)


def _finalize(u2, den2, bias2d):
    bm = 1000
    grid = (N // bm,)
    return pl.pallas_call(
        _fin_body,
        grid=grid,
        in_specs=[
            pl.BlockSpec((1, bm, HID), lambda i: (0, i, 0)),
            pl.BlockSpec((bm, 16), lambda i: (i, 0)),
            pl.BlockSpec((1, HID), lambda i: (0, 0)),
        ],
        out_specs=[
            pl.BlockSpec((bm, HID), lambda i: (i, 0)),
            pl.BlockSpec((bm, 1), lambda i: (i, 0)),
        ],
        out_shape=[
            jax.ShapeDtypeStruct((N, HID), jnp.float32),
            jax.ShapeDtypeStruct((N, 1), jnp.float32),
        ],
    )(u2, den2, bias2d)


def _make_sc_agg(nheads):
    acc_rows = NB + 16
    junk = NB + 8
    stripe = NB // 16

    mesh = plsc.VectorSubcoreMesh(core_axis_name="c", subcore_axis_name="s")

    @functools.partial(
        pl.kernel,
        mesh=mesh,
        compiler_params=pltpu.CompilerParams(
            use_tc_tiling_on_sc=False, needs_layout_passes=False),
        out_type=[
            jax.ShapeDtypeStruct((nheads, OUT_ROWS, HID), jnp.float32),
            jax.ShapeDtypeStruct((OUT_ROWS, 16), jnp.float32),
        ],
        scratch_types=[
            pltpu.VMEM((EWP,), jnp.int32),
            pltpu.VMEM((EWP,), jnp.int32),
            pltpu.VMEM((EWP,), jnp.float32),
            pltpu.VMEM((NPAD,), jnp.float32),
            pltpu.VMEM((NPAD,), jnp.float32),
            pltpu.VMEM((nheads, 16), jnp.float32),
            pltpu.VMEM((MLCAP,), jnp.int32),
            pltpu.VMEM((CH, HID), jnp.float32),
            pltpu.VMEM((CH, 16), jnp.float32),
            pltpu.VMEM((CH,), jnp.int32),
            pltpu.VMEM((CH,), jnp.int32),
            pltpu.VMEM_SHARED((acc_rows, HID), jnp.float32),
            pltpu.VMEM_SHARED((acc_rows, 16), jnp.float32),
            pltpu.SemaphoreType.DMA,
        ],
    )
    def agg(feat_hbm, el_hbm, er_hbm, gmax_hbm, src_hbm, dst_hbm,
            u_hbm, den_hbm,
            src_v, dst_v, p_loc, el_t, er_t, gmax_v, ml, rows_v, prow_buf,
            idx_buf, sidx_buf, acc, acc_p, gsem):
        cid = lax.axis_index("c")
        sid = lax.axis_index("s")
        ebase = sid * EW
        iota16 = lax.iota(jnp.int32, 16)
        zero16 = jnp.zeros((16,), jnp.float32)

        pltpu.sync_copy(src_hbm.at[pl.ds(ebase, EW)], src_v.at[pl.ds(0, EW)])
        pltpu.sync_copy(dst_hbm.at[pl.ds(ebase, EW)], dst_v.at[pl.ds(0, EW)])
        pltpu.sync_copy(gmax_hbm, gmax_v)
        src_v[pl.ds(PAD0, 16)] = jnp.zeros((16,), jnp.int32)
        dst_v[pl.ds(PAD0, 16)] = jnp.full((16,), BIG, jnp.int32)

        for r in range(NPASS):
            base = (2 * r + cid) * NB
            row0 = sid * stripe

            def fill_body(i, c):
                ml[pl.ds(i * 16, 16)] = jnp.full((16,), PAD0, jnp.int32)
                return c
            lax.fori_loop(0, MLCAP // 16, fill_body, 0)

            def ml_body(i, cnt):
                sl = pl.ds(i * 16, 16)
                dl = dst_v[sl] - base
                msk = (dl >= 0) & (dl < NB)
                mi = msk.astype(jnp.int32)
                pos = cnt + plsc.cumsum(mi) - 1
                slotv = i * 16 + iota16
                plsc.store_scatter(ml, [pos], slotv, mask=msk)
                return cnt + jnp.sum(mi)

            cnt = lax.fori_loop(0, NGRP, ml_body, jnp.int32(0))
            trip = (cnt + (CH - 1)) // CH

            for h in range(nheads):
                pltpu.sync_copy(el_hbm.at[h], el_t)
                pltpu.sync_copy(er_hbm.at[h], er_t)
                gm = gmax_v[h, :]

                def p_body(i, c, gm=gm):
                    sl = pl.ds(i * 16, 16)
                    s = src_v[sl]
                    d = jnp.minimum(dst_v[sl], N - 1)
                    va = plsc.load_gather(el_t, [s])
                    vb = plsc.load_gather(er_t, [d])
                    t = va + vb
                    e = jnp.maximum(t, 0.2 * t)
                    t2 = gm + vb
                    vm = jnp.maximum(t2, 0.2 * t2)
                    p_loc[sl] = jnp.exp(e - vm)
                    return c

                lax.fori_loop(0, NGRP, p_body, 0)

                def zrow(j, c):
                    def zcol(k, c2, j=j):
                        rows_v[j, pl.ds(k * 16, 16)] = zero16
                        return c2
                    lax.fori_loop(0, HID // 16, zcol, 0)
                    prow_buf[j, :] = zero16
                    return c
                lax.fori_loop(0, CH, zrow, 0)

                def zacc(t, c):
                    pltpu.sync_copy(rows_v.at[pl.ds(0, 8)],
                                    acc.at[pl.ds(row0 + t * 8, 8)])
                    return c
                lax.fori_loop(0, stripe // 8, zacc, 0)

                if h == 0:
                    def zaccp(t, c):
                        pltpu.sync_copy(prow_buf.at[pl.ds(0, 8)],
                                        acc_p.at[pl.ds(row0 + t * 8, 8)])
                        return c
                    lax.fori_loop(0, stripe // 8, zaccp, 0)

                plsc.subcore_barrier()

                def chunk_body(ci, c, h=h):
                    co = ci * CH
                    s0 = ml[pl.ds(co, 16)]
                    s1 = ml[pl.ds(co + 16, 16)]
                    src0 = plsc.load_gather(src_v, [s0])
                    src1 = plsc.load_gather(src_v, [s1])
                    dl0 = plsc.load_gather(dst_v, [s0]) - base
                    dl1 = plsc.load_gather(dst_v, [s1]) - base
                    ok0 = (dl0 >= 0) & (dl0 < NB)
                    ok1 = (dl1 >= 0) & (dl1 < NB)
                    sidx_buf[pl.ds(0, 16)] = src0
                    sidx_buf[pl.ds(16, 16)] = src1
                    idx_buf[pl.ds(0, 16)] = jnp.where(ok0, dl0, junk)
                    idx_buf[pl.ds(16, 16)] = jnp.where(ok1, dl1, junk)
                    p0 = plsc.load_gather(p_loc, [s0])
                    p1 = plsc.load_gather(p_loc, [s1])
                    pltpu.async_copy(feat_hbm.at[h].at[sidx_buf], rows_v,
                                     gsem).wait()

                    def edge_body(j, c2):
                        lane = jnp.full((16,), j - (j // 16) * 16, jnp.int32)
                        pv = jnp.where(j < 16, p0, p1)
                        pb = pv.at[lane].get(mode="promise_in_bounds")
                        for k in range(HID // 16):
                            col = k * 16
                            seg = rows_v[j, pl.ds(col, 16)]
                            rows_v[j, pl.ds(col, 16)] = seg * pb
                        prow_buf[j, :] = jnp.where(iota16 == h, pb, 0.0)
                        return c2

                    lax.fori_loop(0, CH, edge_body, 0)
                    pltpu.sync_copy(rows_v, acc.at[idx_buf], add=True)
                    pltpu.sync_copy(prow_buf, acc_p.at[idx_buf], add=True)
                    return c

                lax.fori_loop(0, trip, chunk_body, 0)
                plsc.subcore_barrier()

                def drain(t, c, h=h):
                    off = row0 + t * 8
                    pltpu.sync_copy(acc.at[pl.ds(off, 8)],
                                    u_hbm.at[h].at[pl.ds(base + off, 8)])
                    return c
                lax.fori_loop(0, stripe // 8, drain, 0)

                if h == nheads - 1:
                    def draind(t, c):
                        off = row0 + t * 8
                        pltpu.sync_copy(acc_p.at[pl.ds(off, 8)],
                                        den_hbm.at[pl.ds(base + off, 8)])
                        return c
                    lax.fori_loop(0, stripe // 8, draind, 0)

    return agg


@functools.lru_cache(maxsize=None)
def _sc_agg(nheads):
    return _make_sc_agg(nheads)


def kernel(x, W1, attn_l1, attn_r1, bias1, W2, attn_l2, attn_r2, bias2,
           edge_index, graph_len):
    src = edge_index[0].astype(jnp.int32)
    dst = edge_index[1].astype(jnp.int32)
    xp = jnp.pad(x, ((0, NP - N), (0, 0)))

    feat1, el1, er1 = _proj(xp, W1, attn_l1.reshape(1, 4 * HID),
                            attn_r1.reshape(1, 4 * HID), 4)
    g1 = _gmax(el1)
    u1, den1 = _sc_agg(4)(feat1, el1, er1, g1, src, dst)
    h1 = _udiv_relu(u1, den1, bias1.reshape(1, 4 * HID), 4)

    feat2, el2, er2 = _proj(h1, W2, attn_l2.reshape(1, HID),
                            attn_r2.reshape(1, HID), 1)
    g2 = _gmax(el2)
    u2, den2 = _sc_agg(1)(feat2, el2, er2, g2, src, dst)
    graph_output, ge = _finalize(u2, den2, bias2.reshape(1, HID))
    graph_embedding = ge.reshape(N)
    return (graph_embedding, graph_output)


# speedup vs baseline: 12.2694x; 1.2874x over previous
import functools

import jax
import jax.numpy as jnp
from jax import lax
from jax.experimental import pallas as pl
from jax.experimental.pallas import tpu as pltpu
from jax.experimental.pallas import tpu_sc as plsc

N = 10000
E = 160000
HID = 256

EW = E // 16
EWP = 10112
NGRP = (EW + 16) // 16
PAD0 = EW
MLCAP = 10112
NP = 10240
NPAD = 10240
CH = 64
BIG = 1 << 20

NB = 2560
NPASS = 2
OUT_ROWS = 2 * NPASS * NB


def _proj_body(nheads, x_ref, w_ref, al_ref, ar_ref, ft_ref, el_ref, er_ref):
    x = x_ref[...]
    w = w_ref[...]
    feat = jnp.dot(x, w, preferred_element_type=jnp.float32)
    hd = w.shape[1]
    for h in range(nheads):
        ft_ref[h, :, :] = feat[:, h * HID:(h + 1) * HID]
    row = lax.broadcasted_iota(jnp.int32, (nheads, hd), 0)
    col = lax.broadcasted_iota(jnp.int32, (nheads, hd), 1)
    blk = col // HID
    dn = (([1], [1]), ([], []))
    alf = jnp.where(row == blk, jnp.broadcast_to(al_ref[...], (nheads, hd)), 0.0)
    arf = jnp.where(row == blk, jnp.broadcast_to(ar_ref[...], (nheads, hd)), 0.0)
    i = pl.program_id(0)
    bm = x.shape[0]
    el_ref[:, pl.ds(i * bm, bm)] = lax.dot_general(
        alf, feat, dn, preferred_element_type=jnp.float32)
    er_ref[:, pl.ds(i * bm, bm)] = lax.dot_general(
        arf, feat, dn, preferred_element_type=jnp.float32)


def _proj(x, w, attn_l, attn_r, nheads):
    n = x.shape[0]
    hd = w.shape[1]
    bm = 1024
    grid = (n // bm,)
    return pl.pallas_call(
        functools.partial(_proj_body, nheads),
        grid=grid,
        in_specs=[
            pl.BlockSpec((bm, x.shape[1]), lambda i: (i, 0)),
            pl.BlockSpec((x.shape[1], hd), lambda i: (0, 0)),
            pl.BlockSpec((1, hd), lambda i: (0, 0)),
            pl.BlockSpec((1, hd), lambda i: (0, 0)),
        ],
        out_specs=[
            pl.BlockSpec((nheads, bm, HID), lambda i: (0, i, 0)),
            pl.BlockSpec((nheads, n), lambda i: (0, 0)),
            pl.BlockSpec((nheads, n), lambda i: (0, 0)),
        ],
        out_shape=[
            jax.ShapeDtypeStruct((nheads, n, HID), jnp.float32),
            jax.ShapeDtypeStruct((nheads, n), jnp.float32),
            jax.ShapeDtypeStruct((nheads, n), jnp.float32),
        ],
    )(x, w, attn_l, attn_r)


def _mx_body(el_ref, g_ref):
    gmax = jnp.max(el_ref[...], axis=1, keepdims=True)
    g_ref[...] = jnp.broadcast_to(gmax, g_ref.shape)


def _gmax(elT):
    h, n = elT.shape
    return pl.pallas_call(
        _mx_body,
        out_shape=jax.ShapeDtypeStruct((h, 16), jnp.float32),
    )(elT)


def _udiv_body(nheads, u_ref, d_ref, b_ref, h_ref):
    u = u_ref[...]
    d = d_ref[...]
    b = b_ref[...]
    segs = []
    for h in range(nheads):
        dh = d[:, h:h + 1]
        segs.append(jnp.where(dh > 0.0, u[h] / dh, 0.0))
    hh = jnp.concatenate(segs, axis=1) if nheads > 1 else segs[0]
    h_ref[...] = jnp.maximum(hh + b, 0.0)


def _udiv_relu(u_pad, den_pad, bias2d, nheads):
    bm = 1024
    grid = (NP // bm,)
    return pl.pallas_call(
        functools.partial(_udiv_body, nheads),
        grid=grid,
        in_specs=[
            pl.BlockSpec((nheads, bm, HID), lambda i: (0, i, 0)),
            pl.BlockSpec((bm, 16), lambda i: (i, 0)),
            pl.BlockSpec((1, nheads * HID), lambda i: (0, 0)),
        ],
        out_specs=pl.BlockSpec((bm, nheads * HID), lambda i: (i, 0)),
        out_shape=jax.ShapeDtypeStruct((NP, nheads * HID), jnp.float32),
    )(u_pad, den_pad, bias2d)


def _fin_body(u_ref, d_ref, b_ref, g_ref, ge_ref):
    u = u_ref[...][0]
    dh = d_ref[...][:, 0:1]
    g = jnp.where(dh > 0.0, u / dh, 0.0)
    g = jnp.maximum(g + b_ref[...], 0.0)
    g_ref[...] = g
    ge_ref[...] = jnp.sum(g, axis=1, keepdims=True)


def _finalize(u2, den2, bias2d):
    bm = 1000
    grid = (N // bm,)
    return pl.pallas_call(
        _fin_body,
        grid=grid,
        in_specs=[
            pl.BlockSpec((1, bm, HID), lambda i: (0, i, 0)),
            pl.BlockSpec((bm, 16), lambda i: (i, 0)),
            pl.BlockSpec((1, HID), lambda i: (0, 0)),
        ],
        out_specs=[
            pl.BlockSpec((bm, HID), lambda i: (i, 0)),
            pl.BlockSpec((bm, 1), lambda i: (i, 0)),
        ],
        out_shape=[
            jax.ShapeDtypeStruct((N, HID), jnp.float32),
            jax.ShapeDtypeStruct((N, 1), jnp.float32),
        ],
    )(u2, den2, bias2d)


def _make_sc_agg(nheads):
    acc_rows = NB + 16
    junk = NB + 8
    stripe = NB // 16

    mesh = plsc.VectorSubcoreMesh(core_axis_name="c", subcore_axis_name="s")

    @functools.partial(
        pl.kernel,
        mesh=mesh,
        compiler_params=pltpu.CompilerParams(
            use_tc_tiling_on_sc=False, needs_layout_passes=False),
        out_type=[
            jax.ShapeDtypeStruct((nheads, OUT_ROWS, HID), jnp.float32),
            jax.ShapeDtypeStruct((OUT_ROWS, 16), jnp.float32),
        ],
        scratch_types=[
            pltpu.VMEM((EWP,), jnp.int32),
            pltpu.VMEM((EWP,), jnp.int32),
            pltpu.VMEM((EWP,), jnp.float32),
            pltpu.VMEM((NPAD,), jnp.float32),
            pltpu.VMEM((NPAD,), jnp.float32),
            pltpu.VMEM((nheads, 16), jnp.float32),
            pltpu.VMEM((MLCAP,), jnp.int32),
            pltpu.VMEM((CH, HID), jnp.float32),
            pltpu.VMEM((CH, 16), jnp.float32),
            pltpu.VMEM((CH,), jnp.int32),
            pltpu.VMEM((CH,), jnp.int32),
            pltpu.VMEM_SHARED((acc_rows, HID), jnp.float32),
            pltpu.VMEM_SHARED((acc_rows, 16), jnp.float32),
            pltpu.SemaphoreType.DMA,
        ],
    )
    def agg(feat_hbm, el_hbm, er_hbm, gmax_hbm, src_hbm, dst_hbm,
            u_hbm, den_hbm,
            src_v, dst_v, p_loc, el_t, er_t, gmax_v, ml, rows_v, prow_buf,
            idx_buf, sidx_buf, acc, acc_p, gsem):
        cid = lax.axis_index("c")
        sid = lax.axis_index("s")
        ebase = sid * EW
        iota16 = lax.iota(jnp.int32, 16)
        zero16 = jnp.zeros((16,), jnp.float32)

        pltpu.sync_copy(src_hbm.at[pl.ds(ebase, EW)], src_v.at[pl.ds(0, EW)])
        pltpu.sync_copy(dst_hbm.at[pl.ds(ebase, EW)], dst_v.at[pl.ds(0, EW)])
        pltpu.sync_copy(gmax_hbm, gmax_v)
        src_v[pl.ds(PAD0, 16)] = jnp.zeros((16,), jnp.int32)
        dst_v[pl.ds(PAD0, 16)] = jnp.full((16,), BIG, jnp.int32)

        for r in range(NPASS):
            base = (2 * r + cid) * NB
            row0 = sid * stripe

            def fill_body(i, c):
                ml[pl.ds(i * 16, 16)] = jnp.full((16,), PAD0, jnp.int32)
                return c
            lax.fori_loop(0, MLCAP // 16, fill_body, 0)

            def ml_body(i, cnt):
                sl = pl.ds(i * 16, 16)
                dl = dst_v[sl] - base
                msk = (dl >= 0) & (dl < NB)
                mi = msk.astype(jnp.int32)
                pos = cnt + plsc.cumsum(mi) - 1
                slotv = i * 16 + iota16
                plsc.store_scatter(ml, [pos], slotv, mask=msk)
                return cnt + jnp.sum(mi)

            cnt = lax.fori_loop(0, NGRP, ml_body, jnp.int32(0))
            trip = (cnt + (CH - 1)) // CH

            for h in range(nheads):
                pltpu.sync_copy(el_hbm.at[h], el_t)
                pltpu.sync_copy(er_hbm.at[h], er_t)
                gm = gmax_v[h, :]

                def p_body(i, c, gm=gm):
                    sl = pl.ds(i * 16, 16)
                    s = src_v[sl]
                    d = jnp.minimum(dst_v[sl], N - 1)
                    va = plsc.load_gather(el_t, [s])
                    vb = plsc.load_gather(er_t, [d])
                    t = va + vb
                    e = jnp.maximum(t, 0.2 * t)
                    t2 = gm + vb
                    vm = jnp.maximum(t2, 0.2 * t2)
                    p_loc[sl] = jnp.exp(e - vm)
                    return c

                lax.fori_loop(0, NGRP, p_body, 0)

                def zrow(j, c):
                    def zcol(k, c2, j=j):
                        rows_v[j, pl.ds(k * 16, 16)] = zero16
                        return c2
                    lax.fori_loop(0, HID // 16, zcol, 0)
                    prow_buf[j, :] = zero16
                    return c
                lax.fori_loop(0, CH, zrow, 0)

                def zacc(t, c):
                    pltpu.sync_copy(rows_v.at[pl.ds(0, 8)],
                                    acc.at[pl.ds(row0 + t * 8, 8)])
                    return c
                lax.fori_loop(0, stripe // 8, zacc, 0)

                if h == 0:
                    def zaccp(t, c):
                        pltpu.sync_copy(prow_buf.at[pl.ds(0, 8)],
                                        acc_p.at[pl.ds(row0 + t * 8, 8)])
                        return c
                    lax.fori_loop(0, stripe // 8, zaccp, 0)

                plsc.subcore_barrier()

                def chunk_body(ci, c, h=h):
                    co = ci * CH
                    pps = []
                    for g in range(CH // 16):
                        sg = ml[pl.ds(co + 16 * g, 16)]
                        srcg = plsc.load_gather(src_v, [sg])
                        dlg = plsc.load_gather(dst_v, [sg]) - base
                        okg = (dlg >= 0) & (dlg < NB)
                        sidx_buf[pl.ds(16 * g, 16)] = srcg
                        idx_buf[pl.ds(16 * g, 16)] = jnp.where(okg, dlg, junk)
                        pps.append(plsc.load_gather(p_loc, [sg]))
                    pltpu.async_copy(feat_hbm.at[h].at[sidx_buf], rows_v,
                                     gsem).wait()

                    def edge_body(j, c2):
                        lane = jnp.full((16,), j - (j // 16) * 16, jnp.int32)
                        pv = pps[-1]
                        for g in range(CH // 16 - 2, -1, -1):
                            pv = jnp.where(j < 16 * (g + 1), pps[g], pv)
                        pb = pv.at[lane].get(mode="promise_in_bounds")
                        for k in range(HID // 16):
                            col = k * 16
                            seg = rows_v[j, pl.ds(col, 16)]
                            rows_v[j, pl.ds(col, 16)] = seg * pb
                        prow_buf[j, :] = jnp.where(iota16 == h, pb, 0.0)
                        return c2

                    lax.fori_loop(0, CH, edge_body, 0, unroll=2)
                    pltpu.sync_copy(rows_v, acc.at[idx_buf], add=True)
                    pltpu.sync_copy(prow_buf, acc_p.at[idx_buf], add=True)
                    return c

                lax.fori_loop(0, trip, chunk_body, 0)
                plsc.subcore_barrier()

                def drain(t, c, h=h):
                    off = row0 + t * 8
                    pltpu.sync_copy(acc.at[pl.ds(off, 8)],
                                    u_hbm.at[h].at[pl.ds(base + off, 8)])
                    return c
                lax.fori_loop(0, stripe // 8, drain, 0)

                if h == nheads - 1:
                    def draind(t, c):
                        off = row0 + t * 8
                        pltpu.sync_copy(acc_p.at[pl.ds(off, 8)],
                                        den_hbm.at[pl.ds(base + off, 8)])
                        return c
                    lax.fori_loop(0, stripe // 8, draind, 0)

    return agg


@functools.lru_cache(maxsize=None)
def _sc_agg(nheads):
    return _make_sc_agg(nheads)


def kernel(x, W1, attn_l1, attn_r1, bias1, W2, attn_l2, attn_r2, bias2,
           edge_index, graph_len):
    src = edge_index[0].astype(jnp.int32)
    dst = edge_index[1].astype(jnp.int32)
    xp = jnp.pad(x, ((0, NP - N), (0, 0)))

    feat1, el1, er1 = _proj(xp, W1, attn_l1.reshape(1, 4 * HID),
                            attn_r1.reshape(1, 4 * HID), 4)
    g1 = _gmax(el1)
    u1, den1 = _sc_agg(4)(feat1, el1, er1, g1, src, dst)
    h1 = _udiv_relu(u1, den1, bias1.reshape(1, 4 * HID), 4)

    feat2, el2, er2 = _proj(h1, W2, attn_l2.reshape(1, HID),
                            attn_r2.reshape(1, HID), 1)
    g2 = _gmax(el2)
    u2, den2 = _sc_agg(1)(feat2, el2, er2, g2, src, dst)
    graph_output, ge = _finalize(u2, den2, bias2.reshape(1, HID))
    graph_embedding = ge.reshape(N)
    return (graph_embedding, graph_output)


# speedup vs baseline: 12.3836x; 1.0093x over previous
import functools

import jax
import jax.numpy as jnp
from jax import lax
from jax.experimental import pallas as pl
from jax.experimental.pallas import tpu as pltpu
from jax.experimental.pallas import tpu_sc as plsc

N = 10000
E = 160000
HID = 256

EW = E // 16
EWP = 10112
NGRP = (EW + 16) // 16
PAD0 = EW
MLCAP = 10112
NP = 10240
NPAD = 10240
CH = 64
BIG = 1 << 20

NB = 2560
NPASS = 2
OUT_ROWS = 2 * NPASS * NB


def _proj_body(nheads, x_ref, w_ref, al_ref, ar_ref, ft_ref, el_ref, er_ref):
    x = x_ref[...]
    w = w_ref[...]
    feat = jnp.dot(x, w, preferred_element_type=jnp.float32)
    hd = w.shape[1]
    for h in range(nheads):
        ft_ref[h, :, :] = feat[:, h * HID:(h + 1) * HID]
    row = lax.broadcasted_iota(jnp.int32, (nheads, hd), 0)
    col = lax.broadcasted_iota(jnp.int32, (nheads, hd), 1)
    blk = col // HID
    dn = (([1], [1]), ([], []))
    alf = jnp.where(row == blk, jnp.broadcast_to(al_ref[...], (nheads, hd)), 0.0)
    arf = jnp.where(row == blk, jnp.broadcast_to(ar_ref[...], (nheads, hd)), 0.0)
    i = pl.program_id(0)
    bm = x.shape[0]
    el_ref[:, pl.ds(i * bm, bm)] = lax.dot_general(
        alf, feat, dn, preferred_element_type=jnp.float32)
    er_ref[:, pl.ds(i * bm, bm)] = lax.dot_general(
        arf, feat, dn, preferred_element_type=jnp.float32)


def _proj(x, w, attn_l, attn_r, nheads):
    n = x.shape[0]
    hd = w.shape[1]
    bm = 1024
    grid = (n // bm,)
    return pl.pallas_call(
        functools.partial(_proj_body, nheads),
        grid=grid,
        in_specs=[
            pl.BlockSpec((bm, x.shape[1]), lambda i: (i, 0)),
            pl.BlockSpec((x.shape[1], hd), lambda i: (0, 0)),
            pl.BlockSpec((1, hd), lambda i: (0, 0)),
            pl.BlockSpec((1, hd), lambda i: (0, 0)),
        ],
        out_specs=[
            pl.BlockSpec((nheads, bm, HID), lambda i: (0, i, 0)),
            pl.BlockSpec((nheads, n), lambda i: (0, 0)),
            pl.BlockSpec((nheads, n), lambda i: (0, 0)),
        ],
        out_shape=[
            jax.ShapeDtypeStruct((nheads, n, HID), jnp.float32),
            jax.ShapeDtypeStruct((nheads, n), jnp.float32),
            jax.ShapeDtypeStruct((nheads, n), jnp.float32),
        ],
    )(x, w, attn_l, attn_r)


def _mx_body(el_ref, g_ref):
    gmax = jnp.max(el_ref[...], axis=1, keepdims=True)
    g_ref[...] = jnp.broadcast_to(gmax, g_ref.shape)


def _gmax(elT):
    h, n = elT.shape
    return pl.pallas_call(
        _mx_body,
        out_shape=jax.ShapeDtypeStruct((h, 16), jnp.float32),
    )(elT)


def _udiv_body(nheads, u_ref, d_ref, b_ref, h_ref):
    u = u_ref[...]
    d = d_ref[...]
    b = b_ref[...]
    segs = []
    for h in range(nheads):
        dh = d[:, h:h + 1]
        segs.append(jnp.where(dh > 0.0, u[h] / dh, 0.0))
    hh = jnp.concatenate(segs, axis=1) if nheads > 1 else segs[0]
    h_ref[...] = jnp.maximum(hh + b, 0.0)


def _udiv_relu(u_pad, den_pad, bias2d, nheads):
    bm = 1024
    grid = (NP // bm,)
    return pl.pallas_call(
        functools.partial(_udiv_body, nheads),
        grid=grid,
        in_specs=[
            pl.BlockSpec((nheads, bm, HID), lambda i: (0, i, 0)),
            pl.BlockSpec((bm, 16), lambda i: (i, 0)),
            pl.BlockSpec((1, nheads * HID), lambda i: (0, 0)),
        ],
        out_specs=pl.BlockSpec((bm, nheads * HID), lambda i: (i, 0)),
        out_shape=jax.ShapeDtypeStruct((NP, nheads * HID), jnp.float32),
    )(u_pad, den_pad, bias2d)


def _fin_body(u_ref, d_ref, b_ref, g_ref, ge_ref):
    u = u_ref[...][0]
    dh = d_ref[...][:, 0:1]
    g = jnp.where(dh > 0.0, u / dh, 0.0)
    g = jnp.maximum(g + b_ref[...], 0.0)
    g_ref[...] = g
    ge_ref[...] = jnp.sum(g, axis=1, keepdims=True)


def _finalize(u2, den2, bias2d):
    bm = 1000
    grid = (N // bm,)
    return pl.pallas_call(
        _fin_body,
        grid=grid,
        in_specs=[
            pl.BlockSpec((1, bm, HID), lambda i: (0, i, 0)),
            pl.BlockSpec((bm, 16), lambda i: (i, 0)),
            pl.BlockSpec((1, HID), lambda i: (0, 0)),
        ],
        out_specs=[
            pl.BlockSpec((bm, HID), lambda i: (i, 0)),
            pl.BlockSpec((bm, 1), lambda i: (i, 0)),
        ],
        out_shape=[
            jax.ShapeDtypeStruct((N, HID), jnp.float32),
            jax.ShapeDtypeStruct((N, 1), jnp.float32),
        ],
    )(u2, den2, bias2d)


def _make_sc_agg(nheads):
    acc_rows = NB + 16
    junk = NB + 8
    stripe = NB // 16

    mesh = plsc.VectorSubcoreMesh(core_axis_name="c", subcore_axis_name="s")

    @functools.partial(
        pl.kernel,
        mesh=mesh,
        compiler_params=pltpu.CompilerParams(
            use_tc_tiling_on_sc=False, needs_layout_passes=False),
        out_type=[
            jax.ShapeDtypeStruct((nheads, OUT_ROWS, HID), jnp.float32),
            jax.ShapeDtypeStruct((OUT_ROWS, 16), jnp.float32),
        ],
        scratch_types=[
            pltpu.VMEM((EWP,), jnp.int32),
            pltpu.VMEM((EWP,), jnp.int32),
            pltpu.VMEM((EWP,), jnp.float32),
            pltpu.VMEM((NPAD,), jnp.float32),
            pltpu.VMEM((NPAD,), jnp.float32),
            pltpu.VMEM((nheads, 16), jnp.float32),
            pltpu.VMEM((MLCAP,), jnp.int32),
            pltpu.VMEM((CH, HID), jnp.float32),
            pltpu.VMEM((CH, 16), jnp.float32),
            pltpu.VMEM((CH,), jnp.int32),
            pltpu.VMEM((CH,), jnp.int32),
            pltpu.VMEM_SHARED((acc_rows, HID), jnp.float32),
            pltpu.VMEM_SHARED((acc_rows, 16), jnp.float32),
            pltpu.SemaphoreType.DMA,
        ],
    )
    def agg(feat_hbm, el_hbm, er_hbm, gmax_hbm, src_hbm, dst_hbm,
            u_hbm, den_hbm,
            src_v, dst_v, p_loc, el_t, er_t, gmax_v, ml, rows_v, prow_buf,
            idx_buf, sidx_buf, acc, acc_p, gsem):
        cid = lax.axis_index("c")
        sid = lax.axis_index("s")
        ebase = sid * EW
        iota16 = lax.iota(jnp.int32, 16)
        zero16 = jnp.zeros((16,), jnp.float32)

        pltpu.sync_copy(src_hbm.at[pl.ds(ebase, EW)], src_v.at[pl.ds(0, EW)])
        pltpu.sync_copy(dst_hbm.at[pl.ds(ebase, EW)], dst_v.at[pl.ds(0, EW)])
        pltpu.sync_copy(gmax_hbm, gmax_v)
        src_v[pl.ds(PAD0, 16)] = jnp.zeros((16,), jnp.int32)
        dst_v[pl.ds(PAD0, 16)] = jnp.full((16,), BIG, jnp.int32)

        for r in range(NPASS):
            base = (2 * r + cid) * NB
            row0 = sid * stripe

            def fill_body(i, c):
                ml[pl.ds(i * 16, 16)] = jnp.full((16,), PAD0, jnp.int32)
                return c
            lax.fori_loop(0, MLCAP // 16, fill_body, 0)

            def ml_body(i, cnt):
                sl = pl.ds(i * 16, 16)
                dl = dst_v[sl] - base
                msk = (dl >= 0) & (dl < NB)
                mi = msk.astype(jnp.int32)
                pos = cnt + plsc.cumsum(mi) - 1
                slotv = i * 16 + iota16
                plsc.store_scatter(ml, [pos], slotv, mask=msk)
                return cnt + jnp.sum(mi)

            cnt = lax.fori_loop(0, NGRP, ml_body, jnp.int32(0))
            trip = (cnt + (CH - 1)) // CH

            for h in range(nheads):
                pltpu.sync_copy(el_hbm.at[h], el_t)
                pltpu.sync_copy(er_hbm.at[h], er_t)
                gm = gmax_v[h, :]

                def p_body(i, c, gm=gm):
                    sl = pl.ds(i * 16, 16)
                    s = src_v[sl]
                    d = jnp.minimum(dst_v[sl], N - 1)
                    va = plsc.load_gather(el_t, [s])
                    vb = plsc.load_gather(er_t, [d])
                    t = va + vb
                    e = jnp.maximum(t, 0.2 * t)
                    t2 = gm + vb
                    vm = jnp.maximum(t2, 0.2 * t2)
                    p_loc[sl] = jnp.exp(e - vm)
                    return c

                lax.fori_loop(0, NGRP, p_body, 0)

                def zrow(j, c):
                    def zcol(k, c2, j=j):
                        rows_v[j, pl.ds(k * 16, 16)] = zero16
                        return c2
                    lax.fori_loop(0, HID // 16, zcol, 0)
                    prow_buf[j, :] = zero16
                    return c
                lax.fori_loop(0, CH, zrow, 0)

                def zacc(t, c):
                    pltpu.sync_copy(rows_v.at[pl.ds(0, 8)],
                                    acc.at[pl.ds(row0 + t * 8, 8)])
                    return c
                lax.fori_loop(0, stripe // 8, zacc, 0)

                if h == 0:
                    def zaccp(t, c):
                        pltpu.sync_copy(prow_buf.at[pl.ds(0, 8)],
                                        acc_p.at[pl.ds(row0 + t * 8, 8)])
                        return c
                    lax.fori_loop(0, stripe // 8, zaccp, 0)

                plsc.subcore_barrier()

                def chunk_body(ci, c, h=h):
                    co = ci * CH
                    pps = []
                    for g in range(CH // 16):
                        sg = ml[pl.ds(co + 16 * g, 16)]
                        srcg = plsc.load_gather(src_v, [sg])
                        dlg = plsc.load_gather(dst_v, [sg]) - base
                        okg = (dlg >= 0) & (dlg < NB)
                        sidx_buf[pl.ds(16 * g, 16)] = srcg
                        idx_buf[pl.ds(16 * g, 16)] = jnp.where(okg, dlg, junk)
                        pps.append(plsc.load_gather(p_loc, [sg]))
                    pltpu.async_copy(feat_hbm.at[h].at[sidx_buf], rows_v,
                                     gsem).wait()

                    for g in range(CH // 16):
                        pv = pps[g]

                        def edge_body(jj, c2, g=g, pv=pv):
                            j = 16 * g + jj
                            lane = jnp.full((16,), jj, jnp.int32)
                            pb = pv.at[lane].get(mode="promise_in_bounds")
                            for k in range(HID // 16):
                                col = k * 16
                                seg = rows_v[j, pl.ds(col, 16)]
                                rows_v[j, pl.ds(col, 16)] = seg * pb
                            prow_buf[j, :] = jnp.where(iota16 == h, pb, 0.0)
                            return c2

                        lax.fori_loop(0, 16, edge_body, 0, unroll=2)
                    pltpu.sync_copy(rows_v, acc.at[idx_buf], add=True)
                    pltpu.sync_copy(prow_buf, acc_p.at[idx_buf], add=True)
                    return c

                lax.fori_loop(0, trip, chunk_body, 0)
                plsc.subcore_barrier()

                def drain(t, c, h=h):
                    off = row0 + t * 8
                    pltpu.sync_copy(acc.at[pl.ds(off, 8)],
                                    u_hbm.at[h].at[pl.ds(base + off, 8)])
                    return c
                lax.fori_loop(0, stripe // 8, drain, 0)

                if h == nheads - 1:
                    def draind(t, c):
                        off = row0 + t * 8
                        pltpu.sync_copy(acc_p.at[pl.ds(off, 8)],
                                        den_hbm.at[pl.ds(base + off, 8)])
                        return c
                    lax.fori_loop(0, stripe // 8, draind, 0)

    return agg


@functools.lru_cache(maxsize=None)
def _sc_agg(nheads):
    return _make_sc_agg(nheads)


def kernel(x, W1, attn_l1, attn_r1, bias1, W2, attn_l2, attn_r2, bias2,
           edge_index, graph_len):
    src = edge_index[0].astype(jnp.int32)
    dst = edge_index[1].astype(jnp.int32)
    xp = jnp.pad(x, ((0, NP - N), (0, 0)))

    feat1, el1, er1 = _proj(xp, W1, attn_l1.reshape(1, 4 * HID),
                            attn_r1.reshape(1, 4 * HID), 4)
    g1 = _gmax(el1)
    u1, den1 = _sc_agg(4)(feat1, el1, er1, g1, src, dst)
    h1 = _udiv_relu(u1, den1, bias1.reshape(1, 4 * HID), 4)

    feat2, el2, er2 = _proj(h1, W2, attn_l2.reshape(1, HID),
                            attn_r2.reshape(1, HID), 1)
    g2 = _gmax(el2)
    u2, den2 = _sc_agg(1)(feat2, el2, er2, g2, src, dst)
    graph_output, ge = _finalize(u2, den2, bias2.reshape(1, HID))
    graph_embedding = ge.reshape(N)
    return (graph_embedding, graph_output)


# speedup vs baseline: 12.6770x; 1.0237x over previous
import functools

import jax
import jax.numpy as jnp
from jax import lax
from jax.experimental import pallas as pl
from jax.experimental.pallas import tpu as pltpu
from jax.experimental.pallas import tpu_sc as plsc

N = 10000
E = 160000
HID = 256

EW = E // 16
EWP = 10112
NGRP = (EW + 16) // 16
PAD0 = EW
MLCAP = 10240
NP = 10240
NPAD = 10240
CH = 32
BIG = 1 << 20

NB = 2560
NPASS = 2
OUT_ROWS = 2 * NPASS * NB


def _proj_body(nheads, x_ref, w_ref, al_ref, ar_ref, ft_ref, el_ref, er_ref):
    x = x_ref[...]
    w = w_ref[...]
    feat = jnp.dot(x, w, preferred_element_type=jnp.float32)
    hd = w.shape[1]
    for h in range(nheads):
        ft_ref[h, :, :] = feat[:, h * HID:(h + 1) * HID]
    row = lax.broadcasted_iota(jnp.int32, (nheads, hd), 0)
    col = lax.broadcasted_iota(jnp.int32, (nheads, hd), 1)
    blk = col // HID
    dn = (([1], [1]), ([], []))
    alf = jnp.where(row == blk, jnp.broadcast_to(al_ref[...], (nheads, hd)), 0.0)
    arf = jnp.where(row == blk, jnp.broadcast_to(ar_ref[...], (nheads, hd)), 0.0)
    i = pl.program_id(0)
    bm = x.shape[0]
    el_ref[:, pl.ds(i * bm, bm)] = lax.dot_general(
        alf, feat, dn, preferred_element_type=jnp.float32)
    er_ref[:, pl.ds(i * bm, bm)] = lax.dot_general(
        arf, feat, dn, preferred_element_type=jnp.float32)


def _proj(x, w, attn_l, attn_r, nheads):
    n = x.shape[0]
    hd = w.shape[1]
    bm = 1024
    grid = (n // bm,)
    return pl.pallas_call(
        functools.partial(_proj_body, nheads),
        grid=grid,
        in_specs=[
            pl.BlockSpec((bm, x.shape[1]), lambda i: (i, 0)),
            pl.BlockSpec((x.shape[1], hd), lambda i: (0, 0)),
            pl.BlockSpec((1, hd), lambda i: (0, 0)),
            pl.BlockSpec((1, hd), lambda i: (0, 0)),
        ],
        out_specs=[
            pl.BlockSpec((nheads, bm, HID), lambda i: (0, i, 0)),
            pl.BlockSpec((nheads, n), lambda i: (0, 0)),
            pl.BlockSpec((nheads, n), lambda i: (0, 0)),
        ],
        out_shape=[
            jax.ShapeDtypeStruct((nheads, n, HID), jnp.float32),
            jax.ShapeDtypeStruct((nheads, n), jnp.float32),
            jax.ShapeDtypeStruct((nheads, n), jnp.float32),
        ],
    )(x, w, attn_l, attn_r)


def _mx_body(el_ref, g_ref):
    gmax = jnp.max(el_ref[...], axis=1, keepdims=True)
    g_ref[...] = jnp.broadcast_to(gmax, g_ref.shape)


def _gmax(elT):
    h, n = elT.shape
    return pl.pallas_call(
        _mx_body,
        out_shape=jax.ShapeDtypeStruct((h, 16), jnp.float32),
    )(elT)


def _udiv_body(nheads, u_ref, d_ref, b_ref, h_ref):
    u = u_ref[...]
    d = d_ref[...]
    b = b_ref[...]
    segs = []
    for h in range(nheads):
        dh = d[:, h:h + 1]
        segs.append(jnp.where(dh > 0.0, u[h] / dh, 0.0))
    hh = jnp.concatenate(segs, axis=1) if nheads > 1 else segs[0]
    h_ref[...] = jnp.maximum(hh + b, 0.0)


def _udiv_relu(u_pad, den_pad, bias2d, nheads):
    bm = 1024
    grid = (NP // bm,)
    return pl.pallas_call(
        functools.partial(_udiv_body, nheads),
        grid=grid,
        in_specs=[
            pl.BlockSpec((nheads, bm, HID), lambda i: (0, i, 0)),
            pl.BlockSpec((bm, 16), lambda i: (i, 0)),
            pl.BlockSpec((1, nheads * HID), lambda i: (0, 0)),
        ],
        out_specs=pl.BlockSpec((bm, nheads * HID), lambda i: (i, 0)),
        out_shape=jax.ShapeDtypeStruct((NP, nheads * HID), jnp.float32),
    )(u_pad, den_pad, bias2d)


def _fin_body(u_ref, d_ref, b_ref, g_ref, ge_ref):
    u = u_ref[...][0]
    dh = d_ref[...][:, 0:1]
    g = jnp.where(dh > 0.0, u / dh, 0.0)
    g = jnp.maximum(g + b_ref[...], 0.0)
    g_ref[...] = g
    ge_ref[...] = jnp.sum(g, axis=1, keepdims=True)


def _finalize(u2, den2, bias2d):
    bm = 1000
    grid = (N // bm,)
    return pl.pallas_call(
        _fin_body,
        grid=grid,
        in_specs=[
            pl.BlockSpec((1, bm, HID), lambda i: (0, i, 0)),
            pl.BlockSpec((bm, 16), lambda i: (i, 0)),
            pl.BlockSpec((1, HID), lambda i: (0, 0)),
        ],
        out_specs=[
            pl.BlockSpec((bm, HID), lambda i: (i, 0)),
            pl.BlockSpec((bm, 1), lambda i: (i, 0)),
        ],
        out_shape=[
            jax.ShapeDtypeStruct((N, HID), jnp.float32),
            jax.ShapeDtypeStruct((N, 1), jnp.float32),
        ],
    )(u2, den2, bias2d)


def _make_sc_agg(nheads):
    acc_rows = NB + 16
    junk = NB + 8
    stripe = NB // 16

    mesh = plsc.VectorSubcoreMesh(core_axis_name="c", subcore_axis_name="s")

    @functools.partial(
        pl.kernel,
        mesh=mesh,
        compiler_params=pltpu.CompilerParams(
            use_tc_tiling_on_sc=False, needs_layout_passes=False),
        out_type=[
            jax.ShapeDtypeStruct((nheads, OUT_ROWS, HID), jnp.float32),
            jax.ShapeDtypeStruct((OUT_ROWS, 16), jnp.float32),
        ],
        scratch_types=[
            pltpu.VMEM((EWP,), jnp.int32),
            pltpu.VMEM((EWP,), jnp.int32),
            pltpu.VMEM((EWP,), jnp.float32),
            pltpu.VMEM((NPAD,), jnp.float32),
            pltpu.VMEM((NPAD,), jnp.float32),
            pltpu.VMEM((nheads, 16), jnp.float32),
            pltpu.VMEM((MLCAP,), jnp.int32),
            pltpu.VMEM((CH, HID), jnp.float32),
            pltpu.VMEM((CH, HID), jnp.float32),
            pltpu.VMEM((CH, 16), jnp.float32),
            pltpu.VMEM((CH,), jnp.int32),
            pltpu.VMEM((CH,), jnp.int32),
            pltpu.VMEM((CH,), jnp.int32),
            pltpu.VMEM((CH,), jnp.int32),
            pltpu.VMEM_SHARED((acc_rows, HID), jnp.float32),
            pltpu.VMEM_SHARED((acc_rows, 16), jnp.float32),
            pltpu.SemaphoreType.DMA,
            pltpu.SemaphoreType.DMA,
        ],
    )
    def agg(feat_hbm, el_hbm, er_hbm, gmax_hbm, src_hbm, dst_hbm,
            u_hbm, den_hbm,
            src_v, dst_v, p_loc, el_t, er_t, gmax_v, ml, rows_a, rows_b,
            prow_buf, idx_a, idx_b, sidx_a, sidx_b, acc, acc_p,
            gsem_a, gsem_b):
        cid = lax.axis_index("c")
        sid = lax.axis_index("s")
        ebase = sid * EW
        iota16 = lax.iota(jnp.int32, 16)
        zero16 = jnp.zeros((16,), jnp.float32)

        pltpu.sync_copy(src_hbm.at[pl.ds(ebase, EW)], src_v.at[pl.ds(0, EW)])
        pltpu.sync_copy(dst_hbm.at[pl.ds(ebase, EW)], dst_v.at[pl.ds(0, EW)])
        pltpu.sync_copy(gmax_hbm, gmax_v)
        src_v[pl.ds(PAD0, 16)] = jnp.zeros((16,), jnp.int32)
        dst_v[pl.ds(PAD0, 16)] = jnp.full((16,), BIG, jnp.int32)

        for r in range(NPASS):
            base = (2 * r + cid) * NB
            row0 = sid * stripe

            def fill_body(i, c):
                ml[pl.ds(i * 16, 16)] = jnp.full((16,), PAD0, jnp.int32)
                return c
            lax.fori_loop(0, MLCAP // 16, fill_body, 0)

            def ml_body(i, cnt):
                sl = pl.ds(i * 16, 16)
                dl = dst_v[sl] - base
                msk = (dl >= 0) & (dl < NB)
                mi = msk.astype(jnp.int32)
                pos = cnt + plsc.cumsum(mi) - 1
                slotv = i * 16 + iota16
                plsc.store_scatter(ml, [pos], slotv, mask=msk)
                return cnt + jnp.sum(mi)

            cnt = lax.fori_loop(0, NGRP, ml_body, jnp.int32(0))
            trip = (cnt + (CH - 1)) // CH

            for h in range(nheads):
                pltpu.sync_copy(el_hbm.at[h], el_t)
                pltpu.sync_copy(er_hbm.at[h], er_t)
                gm = gmax_v[h, :]

                def p_body(i, c, gm=gm):
                    sl = pl.ds(i * 16, 16)
                    s = src_v[sl]
                    d = jnp.minimum(dst_v[sl], N - 1)
                    va = plsc.load_gather(el_t, [s])
                    vb = plsc.load_gather(er_t, [d])
                    t = va + vb
                    e = jnp.maximum(t, 0.2 * t)
                    t2 = gm + vb
                    vm = jnp.maximum(t2, 0.2 * t2)
                    p_loc[sl] = jnp.exp(e - vm)
                    return c

                lax.fori_loop(0, NGRP, p_body, 0)

                def zrow(j, c):
                    def zcol(k, c2, j=j):
                        rows_a[j, pl.ds(k * 16, 16)] = zero16
                        return c2
                    lax.fori_loop(0, HID // 16, zcol, 0)
                    prow_buf[j, :] = zero16
                    return c
                lax.fori_loop(0, CH, zrow, 0)

                def zacc(t, c):
                    pltpu.sync_copy(rows_a.at[pl.ds(0, 8)],
                                    acc.at[pl.ds(row0 + t * 8, 8)])
                    return c
                lax.fori_loop(0, stripe // 8, zacc, 0)

                if h == 0:
                    def zaccp(t, c):
                        pltpu.sync_copy(prow_buf.at[pl.ds(0, 8)],
                                        acc_p.at[pl.ds(row0 + t * 8, 8)])
                        return c
                    lax.fori_loop(0, stripe // 8, zaccp, 0)

                plsc.subcore_barrier()

                def build_idx(ci, idx_x, sidx_x):
                    co = ci * CH
                    for g in range(CH // 16):
                        sg = ml[pl.ds(co + 16 * g, 16)]
                        srcg = plsc.load_gather(src_v, [sg])
                        dlg = plsc.load_gather(dst_v, [sg]) - base
                        okg = (dlg >= 0) & (dlg < NB)
                        sidx_x[pl.ds(16 * g, 16)] = srcg
                        idx_x[pl.ds(16 * g, 16)] = jnp.where(okg, dlg, junk)

                def proc(ci, rows_x, idx_x, h=h):
                    co = ci * CH
                    for g in range(CH // 16):
                        sg = ml[pl.ds(co + 16 * g, 16)]
                        pv = plsc.load_gather(p_loc, [sg])

                        def edge_body(jj, c2, g=g, pv=pv):
                            j = 16 * g + jj
                            lane = jnp.full((16,), jj, jnp.int32)
                            pb = pv.at[lane].get(mode="promise_in_bounds")
                            for k in range(HID // 16):
                                col = k * 16
                                seg = rows_x[j, pl.ds(col, 16)]
                                rows_x[j, pl.ds(col, 16)] = seg * pb
                            prow_buf[j, :] = jnp.where(iota16 == h, pb, 0.0)
                            return c2

                        lax.fori_loop(0, 16, edge_body, 0, unroll=2)
                    pltpu.sync_copy(rows_x, acc.at[idx_x], add=True)
                    pltpu.sync_copy(prow_buf, acc_p.at[idx_x], add=True)

                def gref(h=h):
                    return feat_hbm.at[h]

                build_idx(0, idx_a, sidx_a)
                pltpu.async_copy(gref().at[sidx_a], rows_a, gsem_a)

                def body2(t, c, h=h):
                    build_idx(2 * t + 1, idx_b, sidx_b)
                    pltpu.async_copy(gref().at[sidx_b], rows_b, gsem_b)
                    pltpu.make_async_copy(gref().at[sidx_a], rows_a,
                                          gsem_a).wait()
                    proc(2 * t, rows_a, idx_a)
                    build_idx(2 * t + 2, idx_a, sidx_a)
                    pltpu.async_copy(gref().at[sidx_a], rows_a, gsem_a)
                    pltpu.make_async_copy(gref().at[sidx_b], rows_b,
                                          gsem_b).wait()
                    proc(2 * t + 1, rows_b, idx_b)
                    return c

                trip2 = (trip + 1) // 2
                lax.fori_loop(0, trip2, body2, 0)
                pltpu.make_async_copy(gref().at[sidx_a], rows_a,
                                      gsem_a).wait()
                plsc.subcore_barrier()

                def drain(t, c, h=h):
                    off = row0 + t * 8
                    pltpu.sync_copy(acc.at[pl.ds(off, 8)],
                                    u_hbm.at[h].at[pl.ds(base + off, 8)])
                    return c
                lax.fori_loop(0, stripe // 8, drain, 0)

                if h == nheads - 1:
                    def draind(t, c):
                        off = row0 + t * 8
                        pltpu.sync_copy(acc_p.at[pl.ds(off, 8)],
                                        den_hbm.at[pl.ds(base + off, 8)])
                        return c
                    lax.fori_loop(0, stripe // 8, draind, 0)

    return agg


@functools.lru_cache(maxsize=None)
def _sc_agg(nheads):
    return _make_sc_agg(nheads)


def kernel(x, W1, attn_l1, attn_r1, bias1, W2, attn_l2, attn_r2, bias2,
           edge_index, graph_len):
    src = edge_index[0].astype(jnp.int32)
    dst = edge_index[1].astype(jnp.int32)
    xp = jnp.pad(x, ((0, NP - N), (0, 0)))

    feat1, el1, er1 = _proj(xp, W1, attn_l1.reshape(1, 4 * HID),
                            attn_r1.reshape(1, 4 * HID), 4)
    g1 = _gmax(el1)
    u1, den1 = _sc_agg(4)(feat1, el1, er1, g1, src, dst)
    h1 = _udiv_relu(u1, den1, bias1.reshape(1, 4 * HID), 4)

    feat2, el2, er2 = _proj(h1, W2, attn_l2.reshape(1, HID),
                            attn_r2.reshape(1, HID), 1)
    g2 = _gmax(el2)
    u2, den2 = _sc_agg(1)(feat2, el2, er2, g2, src, dst)
    graph_output, ge = _finalize(u2, den2, bias2.reshape(1, HID))
    graph_embedding = ge.reshape(N)
    return (graph_embedding, graph_output)
